# baseline (speedup 1.0000x reference)
import functools

import jax
import jax.numpy as jnp
from jax import lax
from jax.experimental import pallas as pl
from jax.experimental.pallas import tpu as pltpu
from jax.experimental.pallas import tpu_sc as plsc

_HID = 64
_SCALE = 8.0
_NC = 2
_NS = 16
_NW = _NC * _NS
_L16 = 16
_CHUNK = 128


def _sc_embed(n_l: int, n_bt: int, vocab_pairs: int):
    chunks = n_l * n_bt
    steps = chunks // _NW
    per_w = steps * _CHUNK
    prep = per_w // _L16
    mesh = plsc.VectorSubcoreMesh(core_axis_name="c", subcore_axis_name="s")

    @functools.partial(
        pl.kernel,
        mesh=mesh,
        out_type=jax.ShapeDtypeStruct((n_l, 8, n_bt, 8, _CHUNK), jnp.float32),
        scratch_types=[
            pltpu.VMEM((per_w,), jnp.int32),
            pltpu.VMEM((per_w,), jnp.int32),
            pltpu.VMEM((2, _CHUNK, _CHUNK), jnp.float32),
            pltpu.VMEM((2, 8, 8, _CHUNK), jnp.float32),
            pltpu.SemaphoreType.DMA,
            pltpu.SemaphoreType.DMA,
            pltpu.SemaphoreType.DMA,
            pltpu.SemaphoreType.DMA,
        ],
        compiler_params=pltpu.CompilerParams(needs_layout_passes=False),
    )
    def k(ids_hbm, tab_hbm, out_hbm, idx_v, pb_v, buf_v, obuf_v,
          g0, g1, s0, s1):
        wid = lax.axis_index("s") * _NC + lax.axis_index("c")
        base = wid * per_w
        gsem = (g0, g1)
        ssem = (s0, s1)

        pltpu.sync_copy(ids_hbm.at[pl.ds(base, per_w)], idx_v)

        def prep_body(i, c):
            sl = pl.ds(i * _L16, _L16)
            v = idx_v[sl]
            pb_v[sl] = lax.shift_left(v, 6) & 64
            idx_v[sl] = lax.shift_right_logical(v, 1)
            return c

        lax.fori_loop(0, prep, prep_body, 0, unroll=4)

        def gstart(s, b):
            pltpu.async_copy(
                tab_hbm.at[idx_v.at[pl.ds(s * _CHUNK, _CHUNK)]],
                buf_v.at[b], gsem[b])

        def gwait(s, b):
            pltpu.make_async_copy(
                tab_hbm.at[idx_v.at[pl.ds(s * _CHUNK, _CHUNK)]],
                buf_v.at[b], gsem[b]).wait()

        def out_slices(s):
            q = wid * steps + s
            return lax.shift_right_logical(q, 7), q & (n_bt - 1)

        def sstart(s, b):
            l, bc = out_slices(s)
            pltpu.async_copy(obuf_v.at[b], out_hbm.at[l, :, bc], ssem[b])

        def swait(s, b):
            l, bc = out_slices(s)
            pltpu.make_async_copy(obuf_v.at[b], out_hbm.at[l, :, bc],
                                  ssem[b]).wait()

        def transpose_scale(s, b):
            src = buf_v.at[b]
            iota = lax.iota(jnp.int32, _L16)

            def hr_body(hr, c):
                h0 = hr * 8
                for bg in range(8):
                    pb = pb_v[pl.ds(s * _CHUNK + bg * _L16, _L16)]
                    rowv = iota + bg * _L16
                    for hs in range(8):
                        colv = pb + (h0 + hs)
                        vals = plsc.load_gather(src, [rowv, colv])
                        obuf_v[b, hr, hs, pl.ds(bg * _L16, _L16)] = (
                            vals * _SCALE)
                return c

            lax.fori_loop(0, 8, hr_body, 0)

        gstart(0, 0)

        def body(g, carry):
            for b in range(2):
                s = g + b
                gwait(s, b)

                @pl.when(s + 1 < steps)
                def _pref(s=s, b=b):
                    gstart(s + 1, 1 - b)

                @pl.when(s >= 2)
                def _drain(s=s, b=b):
                    swait(s - 2, b)

                transpose_scale(s, b)
                sstart(s, b)
            return carry

        lax.fori_loop(0, steps // 2, lambda i, c: body(i * 2, c), 0)
        swait(steps - 2, 0)
        swait(steps - 1, 1)

    return k


def kernel(input_ids, embed_table):
    b, l = input_ids.shape
    vocab, hid = embed_table.shape
    total = b * l
    n_bt = b // _CHUNK
    ids_flat = input_ids.T.reshape(total).astype(jnp.int32)
    tab2 = embed_table.reshape(vocab // 2, 2 * hid)
    out5 = _sc_embed(l, n_bt, vocab // 2)(ids_flat, tab2)
    return out5.transpose(2, 4, 0, 1, 3).reshape(b, l, hid)
